# Initial kernel scaffold; baseline (speedup 1.0000x reference)
import jax
import jax.numpy as jnp
from jax.experimental import pallas as pl


def kernel(x, Wg, bg, W1, b1, W2, b2):
    raise NotImplementedError("write your pallas kernel here")


# speedup vs baseline: 3.0764x; 3.0764x over previous
import jax
import jax.numpy as jnp
from jax import lax
from jax.experimental import pallas as pl
from jax.experimental.pallas import tpu as pltpu

D_MODEL = 1024
D_FF = 4096
E_TOTAL = 8
K_ACTIVE = 2
T_TOKENS = 2048
FF_BLOCK = 1024
N_FF_BLOCKS = D_FF // FF_BLOCK


def _router_body(x_ref, wg_ref, bg_ref, cwt_ref, aux_ref):
    x = x_ref[...]
    wg = wg_ref[...]
    logits = lax.dot_general(x, wg, (((1,), (1,)), ((), ())),
                             preferred_element_type=jnp.float32)
    logits = logits + bg_ref[0, :][None, :]
    m = jnp.max(logits, axis=-1, keepdims=True)
    ex = jnp.exp(logits - m)
    probs = ex / jnp.sum(ex, axis=-1, keepdims=True)

    iota_e = lax.broadcasted_iota(jnp.int32, (T_TOKENS, E_TOTAL), 1)
    p1 = jnp.max(probs, axis=-1, keepdims=True)
    i1 = jnp.min(jnp.where(probs == p1, iota_e, E_TOTAL), axis=-1)
    oh1 = iota_e == i1[:, None]
    probs_m = jnp.where(oh1, -1.0, probs)
    p2 = jnp.max(probs_m, axis=-1, keepdims=True)
    i2 = jnp.min(jnp.where(probs_m == p2, iota_e, E_TOTAL), axis=-1)
    oh2 = iota_e == i2[:, None]
    denom = p1 + p2
    w1 = p1 / denom
    w2 = p2 / denom
    cw = jnp.where(oh1, w1, 0.0) + jnp.where(oh2, w2, 0.0)
    cwt_ref[:, 0, :] = cw.T

    prob_mass = jnp.mean(probs, axis=0)
    counts = jnp.sum(oh1.astype(jnp.float32), axis=0)
    aux = E_TOTAL * jnp.sum(prob_mass * (counts / T_TOKENS))
    aux_ref[...] = jnp.reshape(aux, (1, 1))


def _ffn_body(cwt_ref, x_ref, w1_ref, b1_ref, w2_ref, b2_ref, out_ref):
    e = pl.program_id(0)
    f = pl.program_id(1)

    @pl.when(jnp.logical_and(e == 0, f == 0))
    def _():
        out_ref[...] = jnp.zeros_like(out_ref)

    x = x_ref[...]
    w1 = w1_ref[0]
    h = lax.dot_general(x, w1, (((1,), (1,)), ((), ())),
                        preferred_element_type=jnp.float32)
    h = h + b1_ref[0, 0, 0, :][None, :]
    h = 0.5 * h * (1.0 + lax.erf(h * 0.7071067811865476))

    w2 = w2_ref[0]
    y = lax.dot_general(h, w2, (((1,), (1,)), ((), ())),
                        preferred_element_type=jnp.float32)
    w_e = cwt_ref[0, 0, :][:, None]
    y = jnp.where(f == 0, y + b2_ref[0, 0, :][None, :], y)
    out_ref[...] += y * w_e


def kernel(x, Wg, bg, W1, b1, W2, b2):
    B, S, D = x.shape
    x_flat = x.reshape(-1, D)

    cwt, aux = pl.pallas_call(
        _router_body,
        out_shape=(
            jax.ShapeDtypeStruct((E_TOTAL, 1, T_TOKENS), jnp.float32),
            jax.ShapeDtypeStruct((1, 1), jnp.float32),
        ),
        in_specs=[
            pl.BlockSpec((T_TOKENS, D_MODEL), lambda: (0, 0)),
            pl.BlockSpec((E_TOTAL, D_MODEL), lambda: (0, 0)),
            pl.BlockSpec((1, E_TOTAL), lambda: (0, 0)),
        ],
        out_specs=(
            pl.BlockSpec((E_TOTAL, 1, T_TOKENS), lambda: (0, 0, 0)),
            pl.BlockSpec((1, 1), lambda: (0, 0)),
        ),
    )(x_flat, Wg, bg.reshape(1, E_TOTAL))

    out = pl.pallas_call(
        _ffn_body,
        grid=(E_TOTAL, N_FF_BLOCKS),
        out_shape=jax.ShapeDtypeStruct((T_TOKENS, D_MODEL), jnp.float32),
        in_specs=[
            pl.BlockSpec((1, 1, T_TOKENS), lambda e, f: (e, 0, 0)),
            pl.BlockSpec((T_TOKENS, D_MODEL), lambda e, f: (0, 0)),
            pl.BlockSpec((1, FF_BLOCK, D_MODEL), lambda e, f: (e, f, 0)),
            pl.BlockSpec((1, 1, 1, FF_BLOCK), lambda e, f: (e, f, 0, 0)),
            pl.BlockSpec((1, D_MODEL, FF_BLOCK), lambda e, f: (e, 0, f)),
            pl.BlockSpec((1, 1, D_MODEL), lambda e, f: (e, 0, 0)),
        ],
        out_specs=pl.BlockSpec((T_TOKENS, D_MODEL), lambda e, f: (0, 0)),
    )(cwt, x_flat, W1, b1.reshape(E_TOTAL, N_FF_BLOCKS, 1, FF_BLOCK),
      W2, b2.reshape(E_TOTAL, 1, D_MODEL))

    return out.reshape(B, S, D), aux[0, 0]
